# Initial kernel scaffold; baseline (speedup 1.0000x reference)
import jax
import jax.numpy as jnp
from jax.experimental import pallas as pl


def kernel(g, h, e, lb_delta, ub_delta, snorm_n, snorm_e, label, emb, W0, al0, ar0, gm0, bt0, W1, al1, ar1, gm1, bt1, W2, al2, ar2, gm2, bt2, W3, al3, ar3, gm3, bt3, pW0, pb0, pW1, pb1, pW2, pb2, wW0, wb0, wW1, wb1, wW2, wb2):
    raise NotImplementedError("write your pallas kernel here")


# speedup vs baseline: 41.3045x; 41.3045x over previous
import functools

import numpy as np
import jax
import jax.numpy as jnp
from jax import lax
from jax.experimental import pallas as pl
from jax.experimental.pallas import tpu as pltpu
from jax.experimental.pallas import tpu_sc as plsc

_N = 10000
_E = 320000
_D = 128
_VOCAB = 100
_NC = 2
_NS = 16
_CHUNK = 128
_EPT = 20224
_KCH = _EPT // _CHUNK
_EPAD = _NS * _EPT
_NACC = _N + 112
_RPT = _NACC // _NS
_ZW = 80


def _embed_pre_body(h_ref, emb_ref, w_ref, ab_ref, x_ref, ztab_ref, ert_ref):
    hv = h_ref[...]
    cols = lax.broadcasted_iota(jnp.int32, (_N, _D), 1)
    oh = (cols == hv).astype(jnp.float32)
    x = oh @ emb_ref[...]
    x_ref[...] = x
    z = x @ w_ref[...]
    elr = z @ ab_ref[...]
    el16 = elr[:, :16]
    ztab_ref[:_N, :] = jnp.concatenate([z[:, :64], el16], axis=1)
    ztab_ref[_N:, :] = jnp.concatenate([z[:, 64:], el16], axis=1)
    ert_ref[...] = elr[:, 16:]


def _pre_body(x_ref, w_ref, ab_ref, ztab_ref, ert_ref):
    z = x_ref[...] @ w_ref[...]
    elr = z @ ab_ref[...]
    el16 = elr[:, :16]
    ztab_ref[:_N, :] = jnp.concatenate([z[:, :64], el16], axis=1)
    ztab_ref[_N:, :] = jnp.concatenate([z[:, 64:], el16], axis=1)
    ert_ref[...] = elr[:, 16:]


def _post_body(acc_ref, x_ref, snorm_ref, gm_ref, bt_ref,
               expa_ref, expb_ref, out_ref):
    accA = acc_ref[0]
    accB = acc_ref[1]
    sA = accA[:_N, 64:72]
    sB = accB[:_N, 64:72]
    mA = accA[:_N, :64] * ((1.0 / (sA + 1e-9)) @ expa_ref[...])
    mB = accB[:_N, :64] * ((1.0 / (sB + 1e-9)) @ expb_ref[...])
    out = jnp.concatenate([mA, mB], axis=1) * snorm_ref[...]
    mu = jnp.mean(out, axis=0, keepdims=True)
    ctr = out - mu
    var = jnp.mean(ctr * ctr, axis=0, keepdims=True)
    xn = ctr * lax.rsqrt(var + 1e-5) * gm_ref[...] + bt_ref[...]
    xn = jnp.where(xn > 0.0, xn, jnp.exp(xn) - 1.0)
    out_ref[...] = x_ref[...] + xn


def _final_body(x_ref, label_ref, pw0_ref, pb0_ref, pw1_ref, pb1_ref,
                pw2_ref, pb2_ref, ww0a_ref, ww0b_ref, wb0_ref, ww1_ref,
                wb1_ref, ww2_ref, wb2_ref, lbub_ref,
                p_ref, ghat_ref, w_ref):
    x = x_ref[...]
    a = jnp.maximum(x @ pw0_ref[...] + pb0_ref[...], 0.0)
    a = jnp.maximum(a @ pw1_ref[...] + pb1_ref[...], 0.0)
    p_ref[...] = a @ pw2_ref[...] + pb2_ref[...]
    lab = label_ref[...]
    b = jnp.maximum(x @ ww0a_ref[...] + lab @ ww0b_ref[...] + wb0_ref[...],
                    0.0)
    b = jnp.maximum(b @ ww1_ref[...] + wb1_ref[...], 0.0)
    wl = b @ ww2_ref[...] + wb2_ref[...]
    w = 1.0 / (1.0 + jnp.exp(-wl))
    w_ref[...] = w
    wc = jnp.clip(w, lbub_ref[0:1, 0:1], lbub_ref[0:1, 1:2])
    ghat_ref[...] = (1.0 - wc) * lab + wc * (1.0 / 16.0)


_BCAST_DNUMS = lax.GatherDimensionNumbers(
    offset_dims=(), collapsed_slice_dims=(0,), start_index_map=(0,))


def _bcast_lane(vec, lane):
    idx = jnp.full((16, 1), lane, dtype=jnp.int32)
    return lax.gather(vec, idx, _BCAST_DNUMS, (1,),
                      mode=lax.GatherScatterMode.PROMISE_IN_BOUNDS)


def _make_edge_kernel(n_heads):
    mesh = plsc.VectorSubcoreMesh(core_axis_name="c", subcore_axis_name="s")

    @functools.partial(
        pl.kernel,
        mesh=mesh,
        compiler_params=pltpu.CompilerParams(use_tc_tiling_on_sc=False),
        out_type=jax.ShapeDtypeStruct((_NC, _NACC, _ZW), jnp.float32),
        scratch_types=[
            pltpu.VMEM((_KCH, _CHUNK), jnp.int32),
            pltpu.VMEM((_KCH, _CHUNK), jnp.int32),
            pltpu.VMEM((_CHUNK, _ZW), jnp.float32),
            pltpu.VMEM((_CHUNK, 16), jnp.float32),
            pltpu.VMEM((_CHUNK, _ZW), jnp.float32),
            pltpu.VMEM_SHARED((_NACC, _ZW), jnp.float32),
        ],
    )
    def edge_kernel(src_hbm, dst_hbm, ztab_hbm, ert_hbm, zeros_hbm, out_hbm,
                    src_v, dst_v, zrow_v, err_v, msg_v, acc_sh):
        c = lax.axis_index("c")
        s = lax.axis_index("s")
        pltpu.sync_copy(zeros_hbm.at[pl.ds(s * _RPT, _RPT)],
                        acc_sh.at[pl.ds(s * _RPT, _RPT)])
        pltpu.sync_copy(src_hbm.at[c, s], src_v)
        pltpu.sync_copy(dst_hbm.at[s], dst_v)
        plsc.subcore_barrier()

        def chunk_body(j, carry):
            pltpu.sync_copy(ztab_hbm.at[src_v.at[j]], zrow_v)
            pltpu.sync_copy(ert_hbm.at[dst_v.at[j]], err_v)

            def edge_body(b, cc):
                el = zrow_v[b, pl.ds(64, 16)]
                er = err_v[b, :]
                v = el + er
                ex = jnp.exp(jnp.where(v >= 0.0, v, v * 0.2))
                msg_v[b, pl.ds(64, 16)] = ex
                if n_heads == 1:
                    exb = _bcast_lane(ex, 0)
                    for hh in range(4):
                        msg_v[b, pl.ds(16 * hh, 16)] = (
                            zrow_v[b, pl.ds(16 * hh, 16)] * exb)
                else:
                    for hh in range(4):
                        exb = _bcast_lane(ex, c * 4 + hh)
                        msg_v[b, pl.ds(16 * hh, 16)] = (
                            zrow_v[b, pl.ds(16 * hh, 16)] * exb)
                return cc

            lax.fori_loop(0, _CHUNK, edge_body, 0)
            pltpu.sync_copy(msg_v, acc_sh.at[dst_v.at[j]], add=True)
            return carry

        lax.fori_loop(0, _KCH, chunk_body, 0)
        plsc.subcore_barrier()
        pltpu.sync_copy(acc_sh.at[pl.ds(s * _RPT, _RPT)],
                        out_hbm.at[c, pl.ds(s * _RPT, _RPT)])

    return edge_kernel


def _ab_mat(al, ar):
    h, dh = al.shape
    ab = jnp.zeros((_D, 32), jnp.float32)
    for i in range(h):
        ab = ab.at[i * dh:(i + 1) * dh, i].set(al[i])
        ab = ab.at[i * dh:(i + 1) * dh, 16 + i].set(ar[i])
    return ab


def _exp_mats(n_heads):
    ea = np.zeros((8, 64), np.float32)
    eb = np.zeros((8, 64), np.float32)
    if n_heads == 8:
        for h in range(4):
            ea[h, 16 * h:16 * (h + 1)] = 1.0
            eb[4 + h, 16 * h:16 * (h + 1)] = 1.0
    else:
        ea[0, :] = 1.0
        eb[0, :] = 1.0
    return ea, eb


_EXPA8, _EXPB8 = _exp_mats(8)
_EXPA1, _EXPB1 = _exp_mats(1)


def kernel(g, h, e, lb_delta, ub_delta, snorm_n, snorm_e, label, emb,
           W0, al0, ar0, gm0, bt0, W1, al1, ar1, gm1, bt1,
           W2, al2, ar2, gm2, bt2, W3, al3, ar3, gm3, bt3,
           pW0, pb0, pW1, pb1, pW2, pb2, wW0, wb0, wW1, wb1, wW2, wb2):
    f32 = jnp.float32
    src = g[0].astype(jnp.int32)
    dst = g[1].astype(jnp.int32)
    npad = _EPAD - _E
    ppos = jnp.arange(npad, dtype=jnp.int32)
    src_p = jnp.concatenate([src, (ppos * 37) % _N])
    dst_p = jnp.concatenate([dst, _N + (ppos % (_NACC - _N))])
    src_t = src_p.reshape(_NS, _KCH, _CHUNK)
    src_r = jnp.stack([src_t, src_t + _N])
    dst_r = dst_p.reshape(_NS, _KCH, _CHUNK)
    zeros_acc = jnp.zeros((_NACC, _ZW), f32)

    emb_pad = jnp.zeros((_D, _D), f32).at[:_VOCAB].set(emb)
    h2 = h.astype(jnp.int32).reshape(_N, 1)

    embed_pre = pl.pallas_call(
        _embed_pre_body,
        out_shape=(jax.ShapeDtypeStruct((_N, _D), f32),
                   jax.ShapeDtypeStruct((2 * _N, _ZW), f32),
                   jax.ShapeDtypeStruct((_N, 16), f32)))
    pre = pl.pallas_call(
        _pre_body,
        out_shape=(jax.ShapeDtypeStruct((2 * _N, _ZW), f32),
                   jax.ShapeDtypeStruct((_N, 16), f32)))
    post = pl.pallas_call(
        _post_body,
        out_shape=jax.ShapeDtypeStruct((_N, _D), f32))
    final = pl.pallas_call(
        _final_body,
        out_shape=(jax.ShapeDtypeStruct((_N, 16), f32),
                   jax.ShapeDtypeStruct((_N, 16), f32),
                   jax.ShapeDtypeStruct((_N, 1), f32)))
    edge8 = _make_edge_kernel(8)
    edge1 = _make_edge_kernel(1)

    layers = [(W0, al0, ar0, gm0, bt0, 8), (W1, al1, ar1, gm1, bt1, 8),
              (W2, al2, ar2, gm2, bt2, 8), (W3, al3, ar3, gm3, bt3, 1)]

    x = None
    for li, (Wi, ali, ari, gmi, bti, nh) in enumerate(layers):
        ab = _ab_mat(ali, ari)
        if li == 0:
            x, ztab, ert = embed_pre(h2, emb_pad, Wi, ab)
        else:
            ztab, ert = pre(x, Wi, ab)
        ert_p = jnp.pad(ert, ((0, _NACC - _N), (0, 0)))
        edge = edge8 if nh == 8 else edge1
        accT = edge(src_r, dst_r, ztab, ert_p, zeros_acc)
        ea, eb = (_EXPA8, _EXPB8) if nh == 8 else (_EXPA1, _EXPB1)
        x = post(accT, x, snorm_n.astype(f32), gmi.reshape(1, _D),
                 bti.reshape(1, _D), jnp.asarray(ea), jnp.asarray(eb))

    lbub = jnp.stack([jnp.asarray(lb_delta, f32),
                      jnp.asarray(ub_delta, f32)]).reshape(1, 2)
    p, ghat, w = final(
        x, label.astype(f32), pW0, pb0.reshape(1, -1), pW1, pb1.reshape(1, -1),
        pW2, pb2.reshape(1, -1), wW0[:_D], wW0[_D:], wb0.reshape(1, -1),
        wW1, wb1.reshape(1, -1), wW2, wb2.reshape(1, -1), lbub)
    return (p, ghat, g, w)


# speedup vs baseline: 73.3332x; 1.7754x over previous
import functools

import numpy as np
import jax
import jax.numpy as jnp
from jax import lax
from jax.experimental import pallas as pl
from jax.experimental.pallas import tpu as pltpu
from jax.experimental.pallas import tpu_sc as plsc

_N = 10000
_E = 320000
_D = 128
_VOCAB = 100
_NC = 2
_NS = 16
_CHUNK = 128
_EPT = 20224
_KCH = _EPT // _CHUNK
_EPAD = _NS * _EPT
_NACC = _N + 112
_RPT = _NACC // _NS
_ZW = 80


def _embed_pre_body(h_ref, emb_ref, w_ref, ab_ref, x_ref, ztab_ref, ert_ref):
    hv = h_ref[...]
    cols = lax.broadcasted_iota(jnp.int32, (_N, _D), 1)
    oh = (cols == hv).astype(jnp.float32)
    x = oh @ emb_ref[...]
    x_ref[...] = x
    z = x @ w_ref[...]
    elr = z @ ab_ref[...]
    el16 = elr[:, :16]
    ztab_ref[:_N, :] = jnp.concatenate([z[:, :64], el16], axis=1)
    ztab_ref[_N:, :] = jnp.concatenate([z[:, 64:], el16], axis=1)
    ert_ref[...] = elr[:, 16:]


def _pre_body(x_ref, w_ref, ab_ref, ztab_ref, ert_ref):
    z = x_ref[...] @ w_ref[...]
    elr = z @ ab_ref[...]
    el16 = elr[:, :16]
    ztab_ref[:_N, :] = jnp.concatenate([z[:, :64], el16], axis=1)
    ztab_ref[_N:, :] = jnp.concatenate([z[:, 64:], el16], axis=1)
    ert_ref[...] = elr[:, 16:]


def _post_body(acc_ref, x_ref, snorm_ref, gm_ref, bt_ref,
               expa_ref, expb_ref, out_ref):
    accA = acc_ref[0]
    accB = acc_ref[1]
    sA = accA[:_N, 64:72]
    sB = accB[:_N, 64:72]
    mA = accA[:_N, :64] * ((1.0 / (sA + 1e-9)) @ expa_ref[...])
    mB = accB[:_N, :64] * ((1.0 / (sB + 1e-9)) @ expb_ref[...])
    out = jnp.concatenate([mA, mB], axis=1) * snorm_ref[...]
    mu = jnp.mean(out, axis=0, keepdims=True)
    ctr = out - mu
    var = jnp.mean(ctr * ctr, axis=0, keepdims=True)
    xn = ctr * lax.rsqrt(var + 1e-5) * gm_ref[...] + bt_ref[...]
    xn = jnp.where(xn > 0.0, xn, jnp.exp(xn) - 1.0)
    out_ref[...] = x_ref[...] + xn


def _final_body(x_ref, label_ref, pw0_ref, pb0_ref, pw1_ref, pb1_ref,
                pw2_ref, pb2_ref, ww0a_ref, ww0b_ref, wb0_ref, ww1_ref,
                wb1_ref, ww2_ref, wb2_ref, lbub_ref,
                p_ref, ghat_ref, w_ref):
    x = x_ref[...]
    a = jnp.maximum(x @ pw0_ref[...] + pb0_ref[...], 0.0)
    a = jnp.maximum(a @ pw1_ref[...] + pb1_ref[...], 0.0)
    p_ref[...] = a @ pw2_ref[...] + pb2_ref[...]
    lab = label_ref[...]
    b = jnp.maximum(x @ ww0a_ref[...] + lab @ ww0b_ref[...] + wb0_ref[...],
                    0.0)
    b = jnp.maximum(b @ ww1_ref[...] + wb1_ref[...], 0.0)
    wl = b @ ww2_ref[...] + wb2_ref[...]
    w = 1.0 / (1.0 + jnp.exp(-wl))
    w_ref[...] = w
    wc = jnp.clip(w, lbub_ref[0:1, 0:1], lbub_ref[0:1, 1:2])
    ghat_ref[...] = (1.0 - wc) * lab + wc * (1.0 / 16.0)


_BCAST_DNUMS = lax.GatherDimensionNumbers(
    offset_dims=(), collapsed_slice_dims=(0,), start_index_map=(0,))


def _bcast_lane(vec, lane):
    idx = jnp.full((16, 1), lane, dtype=jnp.int32)
    return lax.gather(vec, idx, _BCAST_DNUMS, (1,),
                      mode=lax.GatherScatterMode.PROMISE_IN_BOUNDS)


def _make_edge_kernel(n_heads):
    mesh = plsc.VectorSubcoreMesh(core_axis_name="c", subcore_axis_name="s")

    @functools.partial(
        pl.kernel,
        mesh=mesh,
        compiler_params=pltpu.CompilerParams(use_tc_tiling_on_sc=False),
        out_type=jax.ShapeDtypeStruct((_NC, _NACC, _ZW), jnp.float32),
        scratch_types=[
            pltpu.VMEM((_KCH, _CHUNK), jnp.int32),
            pltpu.VMEM((4, _CHUNK), jnp.int32),
            pltpu.VMEM((4, _CHUNK), jnp.int32),
            pltpu.VMEM((_CHUNK, _ZW), jnp.float32),
            pltpu.VMEM((_CHUNK, _ZW), jnp.float32),
            pltpu.VMEM((_CHUNK, 16), jnp.float32),
            pltpu.VMEM((_CHUNK, 16), jnp.float32),
            pltpu.VMEM((_CHUNK, _ZW), jnp.float32),
            pltpu.VMEM((_CHUNK, _ZW), jnp.float32),
            pltpu.VMEM_SHARED((_NACC, _ZW), jnp.float32),
            pltpu.SemaphoreType.DMA,
            pltpu.SemaphoreType.DMA,
        ],
    )
    def edge_kernel(pk_hbm, ztab_hbm, ert_hbm, zeros_hbm, out_hbm,
                    pk_v, srci_v, dsti_v, zrow_a, zrow_b, err_a, err_b,
                    msg_a, msg_b, acc_sh, sem_g, sem_s):
        c = lax.axis_index("c")
        s = lax.axis_index("s")
        zrows = (zrow_a, zrow_b)
        errs = (err_a, err_b)
        msgs = (msg_a, msg_b)
        pltpu.sync_copy(zeros_hbm.at[pl.ds(s * _RPT, _RPT)],
                        acc_sh.at[pl.ds(s * _RPT, _RPT)])
        pltpu.sync_copy(pk_hbm.at[c, s], pk_v)
        plsc.subcore_barrier()

        def unpack_idx(j):
            r = lax.rem(j, 4)

            def lane_body(k, cc):
                v = pk_v[j, pl.ds(16 * k, 16)]
                srci_v[r, pl.ds(16 * k, 16)] = lax.shift_right_logical(v, 14)
                dsti_v[r, pl.ds(16 * k, 16)] = jnp.bitwise_and(v, 16383)
                return cc

            lax.fori_loop(0, 8, lane_body, 0, unroll=True)

        def start_gather(j, b):
            r = lax.rem(j, 4)
            pltpu.async_copy(ztab_hbm.at[srci_v.at[r]], zrows[b], sem_g)
            pltpu.async_copy(ert_hbm.at[dsti_v.at[r]], errs[b], sem_g)

        def wait_gather(b):
            pltpu.make_async_copy(ztab_hbm.at[srci_v.at[0]],
                                  zrows[b], sem_g).wait()
            pltpu.make_async_copy(ert_hbm.at[dsti_v.at[0]],
                                  errs[b], sem_g).wait()

        def wait_scatter(b):
            pltpu.make_async_copy(msgs[b],
                                  acc_sh.at[dsti_v.at[0]], sem_s).wait()

        def compute_chunk(j, b):
            zrow_v = zrows[b]
            err_v = errs[b]
            msg_v = msgs[b]

            def edge_body(eb, cc):
                el = zrow_v[eb, pl.ds(64, 16)]
                er = err_v[eb, :]
                v = el + er
                ex = jnp.exp(jnp.where(v >= 0.0, v, v * 0.2))
                msg_v[eb, pl.ds(64, 16)] = ex
                if n_heads == 1:
                    exb = _bcast_lane(ex, 0)
                    for hh in range(4):
                        msg_v[eb, pl.ds(16 * hh, 16)] = (
                            zrow_v[eb, pl.ds(16 * hh, 16)] * exb)
                else:
                    for hh in range(4):
                        exb = _bcast_lane(ex, c * 4 + hh)
                        msg_v[eb, pl.ds(16 * hh, 16)] = (
                            zrow_v[eb, pl.ds(16 * hh, 16)] * exb)
                return cc

            lax.fori_loop(0, _CHUNK, edge_body, 0)

        unpack_idx(0)
        unpack_idx(1)
        start_gather(0, 0)
        start_gather(1, 1)

        def pair_body(i, carry):
            for b in range(2):
                j = 2 * i + b
                pl.when(i > 0)(lambda: wait_scatter(b))
                wait_gather(b)
                compute_chunk(j, b)
                jn = jnp.minimum(j + 2, _KCH - 1)
                unpack_idx(jn)
                start_gather(jn, b)
                pltpu.async_copy(msgs[b], acc_sh.at[dsti_v.at[lax.rem(j, 4)]],
                                 sem_s, add=True)
            return carry

        lax.fori_loop(0, _KCH // 2, pair_body, 0)
        wait_scatter(0)
        wait_scatter(1)
        wait_gather(0)
        wait_gather(1)
        plsc.subcore_barrier()
        pltpu.sync_copy(acc_sh.at[pl.ds(s * _RPT, _RPT)],
                        out_hbm.at[c, pl.ds(s * _RPT, _RPT)])

    return edge_kernel


def _ab_mat(al, ar):
    h, dh = al.shape
    ab = jnp.zeros((_D, 32), jnp.float32)
    for i in range(h):
        ab = ab.at[i * dh:(i + 1) * dh, i].set(al[i])
        ab = ab.at[i * dh:(i + 1) * dh, 16 + i].set(ar[i])
    return ab


def _exp_mats(n_heads):
    ea = np.zeros((8, 64), np.float32)
    eb = np.zeros((8, 64), np.float32)
    if n_heads == 8:
        for h in range(4):
            ea[h, 16 * h:16 * (h + 1)] = 1.0
            eb[4 + h, 16 * h:16 * (h + 1)] = 1.0
    else:
        ea[0, :] = 1.0
        eb[0, :] = 1.0
    return ea, eb


_EXPA8, _EXPB8 = _exp_mats(8)
_EXPA1, _EXPB1 = _exp_mats(1)


def kernel(g, h, e, lb_delta, ub_delta, snorm_n, snorm_e, label, emb,
           W0, al0, ar0, gm0, bt0, W1, al1, ar1, gm1, bt1,
           W2, al2, ar2, gm2, bt2, W3, al3, ar3, gm3, bt3,
           pW0, pb0, pW1, pb1, pW2, pb2, wW0, wb0, wW1, wb1, wW2, wb2):
    f32 = jnp.float32
    src = g[0].astype(jnp.int32)
    dst = g[1].astype(jnp.int32)
    npad = _EPAD - _E
    ppos = jnp.arange(npad, dtype=jnp.int32)
    src_p = jnp.concatenate([src, (ppos * 37) % _N])
    dst_p = jnp.concatenate([dst, _N + (ppos % (_NACC - _N))])
    src_t = src_p.reshape(_NS, _KCH, _CHUNK)
    dst_t = dst_p.reshape(_NS, _KCH, _CHUNK)
    pk_r = jnp.stack([(src_t * 16384 + dst_t),
                      ((src_t + _N) * 16384 + dst_t)])
    zeros_acc = jnp.zeros((_NACC, _ZW), f32)

    emb_pad = jnp.zeros((_D, _D), f32).at[:_VOCAB].set(emb)
    h2 = h.astype(jnp.int32).reshape(_N, 1)

    embed_pre = pl.pallas_call(
        _embed_pre_body,
        out_shape=(jax.ShapeDtypeStruct((_N, _D), f32),
                   jax.ShapeDtypeStruct((2 * _N, _ZW), f32),
                   jax.ShapeDtypeStruct((_N, 16), f32)))
    pre = pl.pallas_call(
        _pre_body,
        out_shape=(jax.ShapeDtypeStruct((2 * _N, _ZW), f32),
                   jax.ShapeDtypeStruct((_N, 16), f32)))
    post = pl.pallas_call(
        _post_body,
        out_shape=jax.ShapeDtypeStruct((_N, _D), f32))
    final = pl.pallas_call(
        _final_body,
        out_shape=(jax.ShapeDtypeStruct((_N, 16), f32),
                   jax.ShapeDtypeStruct((_N, 16), f32),
                   jax.ShapeDtypeStruct((_N, 1), f32)))
    edge8 = _make_edge_kernel(8)
    edge1 = _make_edge_kernel(1)

    layers = [(W0, al0, ar0, gm0, bt0, 8), (W1, al1, ar1, gm1, bt1, 8),
              (W2, al2, ar2, gm2, bt2, 8), (W3, al3, ar3, gm3, bt3, 1)]

    x = None
    for li, (Wi, ali, ari, gmi, bti, nh) in enumerate(layers):
        ab = _ab_mat(ali, ari)
        if li == 0:
            x, ztab, ert = embed_pre(h2, emb_pad, Wi, ab)
        else:
            ztab, ert = pre(x, Wi, ab)
        ert_p = jnp.pad(ert, ((0, _NACC - _N), (0, 0)))
        edge = edge8 if nh == 8 else edge1
        accT = edge(pk_r, ztab, ert_p, zeros_acc)
        ea, eb = (_EXPA8, _EXPB8) if nh == 8 else (_EXPA1, _EXPB1)
        x = post(accT, x, snorm_n.astype(f32), gmi.reshape(1, _D),
                 bti.reshape(1, _D), jnp.asarray(ea), jnp.asarray(eb))

    lbub = jnp.stack([jnp.asarray(lb_delta, f32),
                      jnp.asarray(ub_delta, f32)]).reshape(1, 2)
    p, ghat, w = final(
        x, label.astype(f32), pW0, pb0.reshape(1, -1), pW1, pb1.reshape(1, -1),
        pW2, pb2.reshape(1, -1), wW0[:_D], wW0[_D:], wb0.reshape(1, -1),
        wW1, wb1.reshape(1, -1), wW2, wb2.reshape(1, -1), lbub)
    return (p, ghat, g, w)


# speedup vs baseline: 126.4953x; 1.7249x over previous
import functools

import numpy as np
import jax
import jax.numpy as jnp
from jax import lax
from jax.experimental import pallas as pl
from jax.experimental.pallas import tpu as pltpu
from jax.experimental.pallas import tpu_sc as plsc

_N = 10000
_E = 320000
_D = 128
_VOCAB = 100
_NC = 2
_NS = 16
_CHUNK = 128
_EPT = 20224
_KCH = _EPT // _CHUNK
_EPAD = _NS * _EPT
_NACC = _N + 112
_RPT = _NACC // _NS
_ZW = 80


def _embed_pre_body(h_ref, emb_ref, w_ref, ab_ref, x_ref, ztab_ref, ert_ref):
    hv = h_ref[...]
    cols = lax.broadcasted_iota(jnp.int32, (_N, _D), 1)
    oh = (cols == hv).astype(jnp.float32)
    x = oh @ emb_ref[...]
    x_ref[...] = x
    z = x @ w_ref[...]
    elr = z @ ab_ref[...]
    el16 = elr[:, :16]
    ztab_ref[:_N, :] = jnp.concatenate([z[:, :64], el16], axis=1)
    ztab_ref[_N:, :] = jnp.concatenate([z[:, 64:], el16], axis=1)
    ert_ref[...] = elr[:, 16:]


def _pre_body(x_ref, w_ref, ab_ref, ztab_ref, ert_ref):
    z = x_ref[...] @ w_ref[...]
    elr = z @ ab_ref[...]
    el16 = elr[:, :16]
    ztab_ref[:_N, :] = jnp.concatenate([z[:, :64], el16], axis=1)
    ztab_ref[_N:, :] = jnp.concatenate([z[:, 64:], el16], axis=1)
    ert_ref[...] = elr[:, 16:]


def _post_body(acc_ref, x_ref, snorm_ref, gm_ref, bt_ref,
               expa_ref, expb_ref, out_ref):
    accA = acc_ref[0]
    accB = acc_ref[1]
    sA = accA[:_N, 64:72]
    sB = accB[:_N, 64:72]
    mA = accA[:_N, :64] * ((1.0 / (sA + 1e-9)) @ expa_ref[...])
    mB = accB[:_N, :64] * ((1.0 / (sB + 1e-9)) @ expb_ref[...])
    out = jnp.concatenate([mA, mB], axis=1) * snorm_ref[...]
    mu = jnp.mean(out, axis=0, keepdims=True)
    ctr = out - mu
    var = jnp.mean(ctr * ctr, axis=0, keepdims=True)
    xn = ctr * lax.rsqrt(var + 1e-5) * gm_ref[...] + bt_ref[...]
    xn = jnp.where(xn > 0.0, xn, jnp.exp(xn) - 1.0)
    out_ref[...] = x_ref[...] + xn


def _final_body(x_ref, label_ref, pw0_ref, pb0_ref, pw1_ref, pb1_ref,
                pw2_ref, pb2_ref, ww0a_ref, ww0b_ref, wb0_ref, ww1_ref,
                wb1_ref, ww2_ref, wb2_ref, lbub_ref,
                p_ref, ghat_ref, w_ref):
    x = x_ref[...]
    a = jnp.maximum(x @ pw0_ref[...] + pb0_ref[...], 0.0)
    a = jnp.maximum(a @ pw1_ref[...] + pb1_ref[...], 0.0)
    p_ref[...] = a @ pw2_ref[...] + pb2_ref[...]
    lab = label_ref[...]
    b = jnp.maximum(x @ ww0a_ref[...] + lab @ ww0b_ref[...] + wb0_ref[...],
                    0.0)
    b = jnp.maximum(b @ ww1_ref[...] + wb1_ref[...], 0.0)
    wl = b @ ww2_ref[...] + wb2_ref[...]
    w = 1.0 / (1.0 + jnp.exp(-wl))
    w_ref[...] = w
    wc = jnp.clip(w, lbub_ref[0:1, 0:1], lbub_ref[0:1, 1:2])
    ghat_ref[...] = (1.0 - wc) * lab + wc * (1.0 / 16.0)


_BCAST_DNUMS = lax.GatherDimensionNumbers(
    offset_dims=(), collapsed_slice_dims=(0,), start_index_map=(0,))


def _bcast_lane(vec, lane):
    idx = jnp.full((16, 1), lane, dtype=jnp.int32)
    return lax.gather(vec, idx, _BCAST_DNUMS, (1,),
                      mode=lax.GatherScatterMode.PROMISE_IN_BOUNDS)


def _make_edge_kernel(n_heads):
    mesh = plsc.VectorSubcoreMesh(core_axis_name="c", subcore_axis_name="s")

    @functools.partial(
        pl.kernel,
        mesh=mesh,
        compiler_params=pltpu.CompilerParams(use_tc_tiling_on_sc=False),
        out_type=jax.ShapeDtypeStruct((_NC, _NACC, _ZW), jnp.float32),
        scratch_types=[
            pltpu.VMEM((_KCH, _CHUNK), jnp.int32),
            pltpu.VMEM((4, _CHUNK), jnp.int32),
            pltpu.VMEM((4, _CHUNK), jnp.int32),
            pltpu.VMEM((_CHUNK, _ZW), jnp.float32),
            pltpu.VMEM((_CHUNK, _ZW), jnp.float32),
            pltpu.VMEM((_CHUNK, 16), jnp.float32),
            pltpu.VMEM((_CHUNK, 16), jnp.float32),
            pltpu.VMEM((_CHUNK, _ZW), jnp.float32),
            pltpu.VMEM((_CHUNK, _ZW), jnp.float32),
            pltpu.VMEM_SHARED((_NACC, _ZW), jnp.float32),
            pltpu.SemaphoreType.DMA,
            pltpu.SemaphoreType.DMA,
        ],
    )
    def edge_kernel(pk_hbm, ztab_hbm, ert_hbm, zeros_hbm, out_hbm,
                    pk_v, srci_v, dsti_v, zrow_a, zrow_b, err_a, err_b,
                    msg_a, msg_b, acc_sh, sem_g, sem_s):
        c = lax.axis_index("c")
        s = lax.axis_index("s")
        zrows = (zrow_a, zrow_b)
        errs = (err_a, err_b)
        msgs = (msg_a, msg_b)
        pltpu.sync_copy(zeros_hbm.at[pl.ds(s * _RPT, _RPT)],
                        acc_sh.at[pl.ds(s * _RPT, _RPT)])
        pltpu.sync_copy(pk_hbm.at[c, s], pk_v)
        plsc.subcore_barrier()

        def unpack_idx(j):
            r = lax.rem(j, 4)

            def lane_body(k, cc):
                v = pk_v[j, pl.ds(16 * k, 16)]
                srci_v[r, pl.ds(16 * k, 16)] = lax.shift_right_logical(v, 14)
                dsti_v[r, pl.ds(16 * k, 16)] = jnp.bitwise_and(v, 16383)
                return cc

            lax.fori_loop(0, 8, lane_body, 0, unroll=True)

        def start_gather(j, b):
            r = lax.rem(j, 4)
            pltpu.async_copy(ztab_hbm.at[srci_v.at[r]], zrows[b], sem_g)
            pltpu.async_copy(ert_hbm.at[dsti_v.at[r]], errs[b], sem_g)

        def wait_gather(b):
            pltpu.make_async_copy(ztab_hbm.at[srci_v.at[0]],
                                  zrows[b], sem_g).wait()
            pltpu.make_async_copy(ert_hbm.at[dsti_v.at[0]],
                                  errs[b], sem_g).wait()

        def wait_scatter(b):
            pltpu.make_async_copy(msgs[b],
                                  acc_sh.at[dsti_v.at[0]], sem_s).wait()

        def compute_chunk(j, b):
            zrow_v = zrows[b]
            err_v = errs[b]
            msg_v = msgs[b]

            @plsc.parallel_loop(0, _CHUNK, unroll=4)
            def edge_body(eb):
                el = zrow_v[eb, pl.ds(64, 16)]
                er = err_v[eb, :]
                v = el + er
                ex = jnp.exp(jnp.where(v >= 0.0, v, v * 0.2))
                msg_v[eb, pl.ds(64, 16)] = ex
                if n_heads == 1:
                    exb = _bcast_lane(ex, 0)
                    for hh in range(4):
                        msg_v[eb, pl.ds(16 * hh, 16)] = (
                            zrow_v[eb, pl.ds(16 * hh, 16)] * exb)
                else:
                    for hh in range(4):
                        exb = _bcast_lane(ex, c * 4 + hh)
                        msg_v[eb, pl.ds(16 * hh, 16)] = (
                            zrow_v[eb, pl.ds(16 * hh, 16)] * exb)

        unpack_idx(0)
        unpack_idx(1)
        start_gather(0, 0)
        start_gather(1, 1)

        def pair_body(i, carry):
            for b in range(2):
                j = 2 * i + b
                pl.when(i > 0)(lambda: wait_scatter(b))
                wait_gather(b)
                compute_chunk(j, b)
                jn = jnp.minimum(j + 2, _KCH - 1)
                unpack_idx(jn)
                start_gather(jn, b)
                pltpu.async_copy(msgs[b], acc_sh.at[dsti_v.at[lax.rem(j, 4)]],
                                 sem_s, add=True)
            return carry

        lax.fori_loop(0, _KCH // 2, pair_body, 0)
        wait_scatter(0)
        wait_scatter(1)
        wait_gather(0)
        wait_gather(1)
        plsc.subcore_barrier()
        pltpu.sync_copy(acc_sh.at[pl.ds(s * _RPT, _RPT)],
                        out_hbm.at[c, pl.ds(s * _RPT, _RPT)])

    return edge_kernel


def _ab_mat(al, ar):
    h, dh = al.shape
    ab = jnp.zeros((_D, 32), jnp.float32)
    for i in range(h):
        ab = ab.at[i * dh:(i + 1) * dh, i].set(al[i])
        ab = ab.at[i * dh:(i + 1) * dh, 16 + i].set(ar[i])
    return ab


def _exp_mats(n_heads):
    ea = np.zeros((8, 64), np.float32)
    eb = np.zeros((8, 64), np.float32)
    if n_heads == 8:
        for h in range(4):
            ea[h, 16 * h:16 * (h + 1)] = 1.0
            eb[4 + h, 16 * h:16 * (h + 1)] = 1.0
    else:
        ea[0, :] = 1.0
        eb[0, :] = 1.0
    return ea, eb


_EXPA8, _EXPB8 = _exp_mats(8)
_EXPA1, _EXPB1 = _exp_mats(1)


def kernel(g, h, e, lb_delta, ub_delta, snorm_n, snorm_e, label, emb,
           W0, al0, ar0, gm0, bt0, W1, al1, ar1, gm1, bt1,
           W2, al2, ar2, gm2, bt2, W3, al3, ar3, gm3, bt3,
           pW0, pb0, pW1, pb1, pW2, pb2, wW0, wb0, wW1, wb1, wW2, wb2):
    f32 = jnp.float32
    src = g[0].astype(jnp.int32)
    dst = g[1].astype(jnp.int32)
    npad = _EPAD - _E
    ppos = jnp.arange(npad, dtype=jnp.int32)
    src_p = jnp.concatenate([src, (ppos * 37) % _N])
    dst_p = jnp.concatenate([dst, _N + (ppos % (_NACC - _N))])
    src_t = src_p.reshape(_NS, _KCH, _CHUNK)
    dst_t = dst_p.reshape(_NS, _KCH, _CHUNK)
    pk_r = jnp.stack([(src_t * 16384 + dst_t),
                      ((src_t + _N) * 16384 + dst_t)])
    zeros_acc = jnp.zeros((_NACC, _ZW), f32)

    emb_pad = jnp.zeros((_D, _D), f32).at[:_VOCAB].set(emb)
    h2 = h.astype(jnp.int32).reshape(_N, 1)

    embed_pre = pl.pallas_call(
        _embed_pre_body,
        out_shape=(jax.ShapeDtypeStruct((_N, _D), f32),
                   jax.ShapeDtypeStruct((2 * _N, _ZW), f32),
                   jax.ShapeDtypeStruct((_N, 16), f32)))
    pre = pl.pallas_call(
        _pre_body,
        out_shape=(jax.ShapeDtypeStruct((2 * _N, _ZW), f32),
                   jax.ShapeDtypeStruct((_N, 16), f32)))
    post = pl.pallas_call(
        _post_body,
        out_shape=jax.ShapeDtypeStruct((_N, _D), f32))
    final = pl.pallas_call(
        _final_body,
        out_shape=(jax.ShapeDtypeStruct((_N, 16), f32),
                   jax.ShapeDtypeStruct((_N, 16), f32),
                   jax.ShapeDtypeStruct((_N, 1), f32)))
    edge8 = _make_edge_kernel(8)
    edge1 = _make_edge_kernel(1)

    layers = [(W0, al0, ar0, gm0, bt0, 8), (W1, al1, ar1, gm1, bt1, 8),
              (W2, al2, ar2, gm2, bt2, 8), (W3, al3, ar3, gm3, bt3, 1)]

    x = None
    for li, (Wi, ali, ari, gmi, bti, nh) in enumerate(layers):
        ab = _ab_mat(ali, ari)
        if li == 0:
            x, ztab, ert = embed_pre(h2, emb_pad, Wi, ab)
        else:
            ztab, ert = pre(x, Wi, ab)
        ert_p = jnp.pad(ert, ((0, _NACC - _N), (0, 0)))
        edge = edge8 if nh == 8 else edge1
        accT = edge(pk_r, ztab, ert_p, zeros_acc)
        ea, eb = (_EXPA8, _EXPB8) if nh == 8 else (_EXPA1, _EXPB1)
        x = post(accT, x, snorm_n.astype(f32), gmi.reshape(1, _D),
                 bti.reshape(1, _D), jnp.asarray(ea), jnp.asarray(eb))

    lbub = jnp.stack([jnp.asarray(lb_delta, f32),
                      jnp.asarray(ub_delta, f32)]).reshape(1, 2)
    p, ghat, w = final(
        x, label.astype(f32), pW0, pb0.reshape(1, -1), pW1, pb1.reshape(1, -1),
        pW2, pb2.reshape(1, -1), wW0[:_D], wW0[_D:], wb0.reshape(1, -1),
        wW1, wb1.reshape(1, -1), wW2, wb2.reshape(1, -1), lbub)
    return (p, ghat, g, w)


# speedup vs baseline: 127.0992x; 1.0048x over previous
import functools

import numpy as np
import jax
import jax.numpy as jnp
from jax import lax
from jax.experimental import pallas as pl
from jax.experimental.pallas import tpu as pltpu
from jax.experimental.pallas import tpu_sc as plsc

_N = 10000
_E = 320000
_D = 128
_VOCAB = 100
_NC = 2
_NS = 16
_CHUNK = 128
_EPT = 20224
_KCH = _EPT // _CHUNK
_EPAD = _NS * _EPT
_NACC = _N + 112
_RPT = _NACC // _NS
_ZW = 80


def _embed_pre_body(h_ref, emb_ref, w_ref, ab_ref, x_ref, ztab_ref, ert_ref):
    hv = h_ref[...]
    cols = lax.broadcasted_iota(jnp.int32, (_N, _D), 1)
    oh = (cols == hv).astype(jnp.float32)
    x = oh @ emb_ref[...]
    x_ref[...] = x
    z = x @ w_ref[...]
    elr = z @ ab_ref[...]
    el16 = elr[:, :16]
    ztab_ref[:_N, :] = jnp.concatenate([z[:, :64], el16], axis=1)
    ztab_ref[_N:, :] = jnp.concatenate([z[:, 64:], el16], axis=1)
    ert_ref[...] = elr[:, 16:]


def _pre_body(x_ref, w_ref, ab_ref, ztab_ref, ert_ref):
    z = x_ref[...] @ w_ref[...]
    elr = z @ ab_ref[...]
    el16 = elr[:, :16]
    ztab_ref[:_N, :] = jnp.concatenate([z[:, :64], el16], axis=1)
    ztab_ref[_N:, :] = jnp.concatenate([z[:, 64:], el16], axis=1)
    ert_ref[...] = elr[:, 16:]


def _post_body(acc_ref, x_ref, snorm_ref, gm_ref, bt_ref,
               expa_ref, expb_ref, out_ref):
    accA = acc_ref[0]
    accB = acc_ref[1]
    sA = accA[:_N, 64:72]
    sB = accB[:_N, 64:72]
    mA = accA[:_N, :64] * ((1.0 / (sA + 1e-9)) @ expa_ref[...])
    mB = accB[:_N, :64] * ((1.0 / (sB + 1e-9)) @ expb_ref[...])
    out = jnp.concatenate([mA, mB], axis=1) * snorm_ref[...]
    mu = jnp.mean(out, axis=0, keepdims=True)
    ctr = out - mu
    var = jnp.mean(ctr * ctr, axis=0, keepdims=True)
    xn = ctr * lax.rsqrt(var + 1e-5) * gm_ref[...] + bt_ref[...]
    xn = jnp.where(xn > 0.0, xn, jnp.exp(xn) - 1.0)
    out_ref[...] = x_ref[...] + xn


def _final_body(x_ref, label_ref, pw0_ref, pb0_ref, pw1_ref, pb1_ref,
                pw2_ref, pb2_ref, ww0a_ref, ww0b_ref, wb0_ref, ww1_ref,
                wb1_ref, ww2_ref, wb2_ref, lbub_ref,
                p_ref, ghat_ref, w_ref):
    x = x_ref[...]
    a = jnp.maximum(x @ pw0_ref[...] + pb0_ref[...], 0.0)
    a = jnp.maximum(a @ pw1_ref[...] + pb1_ref[...], 0.0)
    p_ref[...] = a @ pw2_ref[...] + pb2_ref[...]
    lab = label_ref[...]
    b = jnp.maximum(x @ ww0a_ref[...] + lab @ ww0b_ref[...] + wb0_ref[...],
                    0.0)
    b = jnp.maximum(b @ ww1_ref[...] + wb1_ref[...], 0.0)
    wl = b @ ww2_ref[...] + wb2_ref[...]
    w = 1.0 / (1.0 + jnp.exp(-wl))
    w_ref[...] = w
    wc = jnp.clip(w, lbub_ref[0:1, 0:1], lbub_ref[0:1, 1:2])
    ghat_ref[...] = (1.0 - wc) * lab + wc * (1.0 / 16.0)


_BCAST_DNUMS = lax.GatherDimensionNumbers(
    offset_dims=(), collapsed_slice_dims=(0,), start_index_map=(0,))


def _bcast_lane(vec, lane):
    idx = jnp.full((16, 1), lane, dtype=jnp.int32)
    return lax.gather(vec, idx, _BCAST_DNUMS, (1,),
                      mode=lax.GatherScatterMode.PROMISE_IN_BOUNDS)


def _make_edge_kernel(n_heads):
    mesh = plsc.VectorSubcoreMesh(core_axis_name="c", subcore_axis_name="s")

    @functools.partial(
        pl.kernel,
        mesh=mesh,
        compiler_params=pltpu.CompilerParams(use_tc_tiling_on_sc=False),
        out_type=jax.ShapeDtypeStruct((_NC, _NACC, _ZW), jnp.float32),
        scratch_types=[
            pltpu.VMEM((_KCH, _CHUNK), jnp.int32),
            pltpu.VMEM((4, _CHUNK), jnp.int32),
            pltpu.VMEM((4, _CHUNK), jnp.int32),
            pltpu.VMEM((_CHUNK, _ZW), jnp.float32),
            pltpu.VMEM((_CHUNK, _ZW), jnp.float32),
            pltpu.VMEM((_CHUNK, 16), jnp.float32),
            pltpu.VMEM((_CHUNK, 16), jnp.float32),
            pltpu.VMEM((_CHUNK, _ZW), jnp.float32),
            pltpu.VMEM((_CHUNK, _ZW), jnp.float32),
            pltpu.VMEM_SHARED((_NACC, _ZW), jnp.float32),
            pltpu.SemaphoreType.DMA,
            pltpu.SemaphoreType.DMA,
        ],
    )
    def edge_kernel(pk_hbm, ztab_hbm, ert_hbm, zeros_hbm, out_hbm,
                    pk_v, srci_v, dsti_v, zrow_a, zrow_b, err_a, err_b,
                    msg_a, msg_b, acc_sh, sem_g, sem_s):
        c = lax.axis_index("c")
        s = lax.axis_index("s")
        zrows = (zrow_a, zrow_b)
        errs = (err_a, err_b)
        msgs = (msg_a, msg_b)
        pltpu.sync_copy(zeros_hbm.at[pl.ds(s * _RPT, _RPT)],
                        acc_sh.at[pl.ds(s * _RPT, _RPT)])
        pltpu.sync_copy(pk_hbm.at[c, s], pk_v)
        plsc.subcore_barrier()

        def unpack_idx(j):
            r = lax.rem(j, 4)

            def lane_body(k, cc):
                v = pk_v[j, pl.ds(16 * k, 16)]
                srci_v[r, pl.ds(16 * k, 16)] = lax.shift_right_logical(v, 14)
                dsti_v[r, pl.ds(16 * k, 16)] = jnp.bitwise_and(v, 16383)
                return cc

            lax.fori_loop(0, 8, lane_body, 0, unroll=True)

        def start_gather(j, b):
            r = lax.rem(j, 4)
            pltpu.async_copy(ztab_hbm.at[srci_v.at[r]], zrows[b], sem_g)
            pltpu.async_copy(ert_hbm.at[dsti_v.at[r]], errs[b], sem_g)

        def wait_gather(b):
            pltpu.make_async_copy(ztab_hbm.at[srci_v.at[0]],
                                  zrows[b], sem_g).wait()
            pltpu.make_async_copy(ert_hbm.at[dsti_v.at[0]],
                                  errs[b], sem_g).wait()

        def wait_scatter(b):
            pltpu.make_async_copy(msgs[b],
                                  acc_sh.at[dsti_v.at[0]], sem_s).wait()

        def compute_chunk(j, b):
            zrow_v = zrows[b]
            err_v = errs[b]
            msg_v = msgs[b]

            @plsc.parallel_loop(0, _CHUNK, unroll=8)
            def edge_body(eb):
                el = zrow_v[eb, pl.ds(64, 16)]
                er = err_v[eb, :]
                v = el + er
                ex = jnp.exp(jnp.where(v >= 0.0, v, v * 0.2))
                msg_v[eb, pl.ds(64, 16)] = ex
                if n_heads == 1:
                    exb = _bcast_lane(ex, 0)
                    for hh in range(4):
                        msg_v[eb, pl.ds(16 * hh, 16)] = (
                            zrow_v[eb, pl.ds(16 * hh, 16)] * exb)
                else:
                    for hh in range(4):
                        exb = _bcast_lane(ex, c * 4 + hh)
                        msg_v[eb, pl.ds(16 * hh, 16)] = (
                            zrow_v[eb, pl.ds(16 * hh, 16)] * exb)

        unpack_idx(0)
        unpack_idx(1)
        start_gather(0, 0)
        start_gather(1, 1)

        def pair_body(i, carry):
            for b in range(2):
                j = 2 * i + b
                pl.when(i > 0)(lambda: wait_scatter(b))
                wait_gather(b)
                compute_chunk(j, b)
                jn = jnp.minimum(j + 2, _KCH - 1)
                unpack_idx(jn)
                start_gather(jn, b)
                pltpu.async_copy(msgs[b], acc_sh.at[dsti_v.at[lax.rem(j, 4)]],
                                 sem_s, add=True)
            return carry

        lax.fori_loop(0, _KCH // 2, pair_body, 0)
        wait_scatter(0)
        wait_scatter(1)
        wait_gather(0)
        wait_gather(1)
        plsc.subcore_barrier()
        pltpu.sync_copy(acc_sh.at[pl.ds(s * _RPT, _RPT)],
                        out_hbm.at[c, pl.ds(s * _RPT, _RPT)])

    return edge_kernel


def _ab_mat(al, ar):
    h, dh = al.shape
    ab = jnp.zeros((_D, 32), jnp.float32)
    for i in range(h):
        ab = ab.at[i * dh:(i + 1) * dh, i].set(al[i])
        ab = ab.at[i * dh:(i + 1) * dh, 16 + i].set(ar[i])
    return ab


def _exp_mats(n_heads):
    ea = np.zeros((8, 64), np.float32)
    eb = np.zeros((8, 64), np.float32)
    if n_heads == 8:
        for h in range(4):
            ea[h, 16 * h:16 * (h + 1)] = 1.0
            eb[4 + h, 16 * h:16 * (h + 1)] = 1.0
    else:
        ea[0, :] = 1.0
        eb[0, :] = 1.0
    return ea, eb


_EXPA8, _EXPB8 = _exp_mats(8)
_EXPA1, _EXPB1 = _exp_mats(1)


def kernel(g, h, e, lb_delta, ub_delta, snorm_n, snorm_e, label, emb,
           W0, al0, ar0, gm0, bt0, W1, al1, ar1, gm1, bt1,
           W2, al2, ar2, gm2, bt2, W3, al3, ar3, gm3, bt3,
           pW0, pb0, pW1, pb1, pW2, pb2, wW0, wb0, wW1, wb1, wW2, wb2):
    f32 = jnp.float32
    src = g[0].astype(jnp.int32)
    dst = g[1].astype(jnp.int32)
    npad = _EPAD - _E
    ppos = jnp.arange(npad, dtype=jnp.int32)
    src_p = jnp.concatenate([src, (ppos * 37) % _N])
    dst_p = jnp.concatenate([dst, _N + (ppos % (_NACC - _N))])
    src_t = src_p.reshape(_NS, _KCH, _CHUNK)
    dst_t = dst_p.reshape(_NS, _KCH, _CHUNK)
    pk_r = jnp.stack([(src_t * 16384 + dst_t),
                      ((src_t + _N) * 16384 + dst_t)])
    zeros_acc = jnp.zeros((_NACC, _ZW), f32)

    emb_pad = jnp.zeros((_D, _D), f32).at[:_VOCAB].set(emb)
    h2 = h.astype(jnp.int32).reshape(_N, 1)

    embed_pre = pl.pallas_call(
        _embed_pre_body,
        out_shape=(jax.ShapeDtypeStruct((_N, _D), f32),
                   jax.ShapeDtypeStruct((2 * _N, _ZW), f32),
                   jax.ShapeDtypeStruct((_N, 16), f32)))
    pre = pl.pallas_call(
        _pre_body,
        out_shape=(jax.ShapeDtypeStruct((2 * _N, _ZW), f32),
                   jax.ShapeDtypeStruct((_N, 16), f32)))
    post = pl.pallas_call(
        _post_body,
        out_shape=jax.ShapeDtypeStruct((_N, _D), f32))
    final = pl.pallas_call(
        _final_body,
        out_shape=(jax.ShapeDtypeStruct((_N, 16), f32),
                   jax.ShapeDtypeStruct((_N, 16), f32),
                   jax.ShapeDtypeStruct((_N, 1), f32)))
    edge8 = _make_edge_kernel(8)
    edge1 = _make_edge_kernel(1)

    layers = [(W0, al0, ar0, gm0, bt0, 8), (W1, al1, ar1, gm1, bt1, 8),
              (W2, al2, ar2, gm2, bt2, 8), (W3, al3, ar3, gm3, bt3, 1)]

    x = None
    for li, (Wi, ali, ari, gmi, bti, nh) in enumerate(layers):
        ab = _ab_mat(ali, ari)
        if li == 0:
            x, ztab, ert = embed_pre(h2, emb_pad, Wi, ab)
        else:
            ztab, ert = pre(x, Wi, ab)
        ert_p = jnp.pad(ert, ((0, _NACC - _N), (0, 0)))
        edge = edge8 if nh == 8 else edge1
        accT = edge(pk_r, ztab, ert_p, zeros_acc)
        ea, eb = (_EXPA8, _EXPB8) if nh == 8 else (_EXPA1, _EXPB1)
        x = post(accT, x, snorm_n.astype(f32), gmi.reshape(1, _D),
                 bti.reshape(1, _D), jnp.asarray(ea), jnp.asarray(eb))

    lbub = jnp.stack([jnp.asarray(lb_delta, f32),
                      jnp.asarray(ub_delta, f32)]).reshape(1, 2)
    p, ghat, w = final(
        x, label.astype(f32), pW0, pb0.reshape(1, -1), pW1, pb1.reshape(1, -1),
        pW2, pb2.reshape(1, -1), wW0[:_D], wW0[_D:], wb0.reshape(1, -1),
        wW1, wb1.reshape(1, -1), wW2, wb2.reshape(1, -1), lbub)
    return (p, ghat, g, w)
